# baseline (speedup 1.0000x reference)
import functools

import jax
import jax.numpy as jnp
from jax import lax
from jax.experimental import pallas as pl
from jax.experimental.pallas import tpu as pltpu
from jax.experimental.pallas import tpu_sc as plsc

H = 128
DEPTH = 5
NC = 2
NS = 16
NW = NC * NS


def _mm_body(x_ref, w_ref, o_ref):
    o_ref[...] = jnp.dot(x_ref[...], w_ref[...],
                         preferred_element_type=jnp.float32)


def _matmul(x, w, blk):
    m, k = x.shape
    n = w.shape[1]
    return pl.pallas_call(
        _mm_body,
        grid=(m // blk,),
        in_specs=[
            pl.BlockSpec((blk, k), lambda i: (i, 0)),
            pl.BlockSpec((k, n), lambda i: (0, 0)),
        ],
        out_specs=pl.BlockSpec((blk, n), lambda i: (i, 0)),
        out_shape=jax.ShapeDtypeStruct((m, n), jnp.float32),
    )(x, w)


def _mm_relu_body(x_ref, w_ref, inp_ref, msg_ref):
    acc = jnp.dot(x_ref[...], w_ref[...], preferred_element_type=jnp.float32)
    inp_ref[...] = acc
    msg_ref[...] = jnp.maximum(acc, 0.0)


def _input_matmul(f_bonds, W_i, blk):
    m, k = f_bonds.shape
    n = W_i.shape[1]
    shp = jax.ShapeDtypeStruct((m, n), jnp.float32)
    return pl.pallas_call(
        _mm_relu_body,
        grid=(m // blk,),
        in_specs=[
            pl.BlockSpec((blk, k), lambda i: (i, 0)),
            pl.BlockSpec((k, n), lambda i: (0, 0)),
        ],
        out_specs=[
            pl.BlockSpec((blk, n), lambda i: (i, 0)),
            pl.BlockSpec((blk, n), lambda i: (i, 0)),
        ],
        out_shape=[shp, shp],
    )(f_bonds, W_i)


def _fuse_body(inp_ref, pre_ref, w_ref, msg_ref, h_ref):
    m = jnp.maximum(inp_ref[...] + pre_ref[...], 0.0)
    msg_ref[...] = m
    h_ref[...] = jnp.dot(m, w_ref[...], preferred_element_type=jnp.float32)


def _fuse(inp, pre, w, blk):
    m, n = inp.shape
    shp = jax.ShapeDtypeStruct((m, n), jnp.float32)
    return pl.pallas_call(
        _fuse_body,
        grid=(m // blk,),
        in_specs=[
            pl.BlockSpec((blk, n), lambda i: (i, 0)),
            pl.BlockSpec((blk, n), lambda i: (i, 0)),
            pl.BlockSpec((n, n), lambda i: (0, 0)),
        ],
        out_specs=[
            pl.BlockSpec((blk, n), lambda i: (i, 0)),
            pl.BlockSpec((blk, n), lambda i: (i, 0)),
        ],
        out_shape=[shp, shp],
    )(inp, pre, w)


def _fuse_last_body(inp_ref, pre_ref, msg_ref):
    msg_ref[...] = jnp.maximum(inp_ref[...] + pre_ref[...], 0.0)


def _fuse_last(inp, pre, blk):
    m, n = inp.shape
    return pl.pallas_call(
        _fuse_last_body,
        grid=(m // blk,),
        in_specs=[
            pl.BlockSpec((blk, n), lambda i: (i, 0)),
            pl.BlockSpec((blk, n), lambda i: (i, 0)),
        ],
        out_specs=pl.BlockSpec((blk, n), lambda i: (i, 0)),
        out_shape=jax.ShapeDtypeStruct((m, n), jnp.float32),
    )(inp, pre)


MOLP = 512


def _readout_body(fa_ref, am_ref, ids_ref, wo1_ref, wo2_ref, bo_ref,
                  out_ref, cnt_ref):
    i = pl.program_id(0)
    hidden = jnp.maximum(
        jnp.dot(fa_ref[...], wo1_ref[...], preferred_element_type=jnp.float32)
        + jnp.dot(am_ref[...], wo2_ref[...],
                  preferred_element_type=jnp.float32)
        + bo_ref[...],
        0.0,
    )
    ids = ids_ref[0, 0, :]
    onehot = (lax.broadcasted_iota(jnp.int32, (MOLP, ids.shape[0]), 0)
              == ids[None, :]).astype(jnp.float32)
    part = jnp.dot(onehot, hidden, preferred_element_type=jnp.float32)
    cpart = jnp.sum(onehot, axis=1, keepdims=True)

    @pl.when(i == 0)
    def _():
        out_ref[...] = jnp.zeros_like(out_ref)
        cnt_ref[...] = jnp.zeros_like(cnt_ref)

    out_ref[...] += part
    cnt_ref[...] += jnp.broadcast_to(cpart, cnt_ref.shape)

    @pl.when(i == pl.num_programs(0) - 1)
    def _():
        out_ref[...] = out_ref[...] / jnp.maximum(cnt_ref[...], 1.0)


def _readout(f_atoms, a_msg, ids3, Wo1, Wo2, b_o, blk):
    na, fa = f_atoms.shape
    grid = na // blk
    return pl.pallas_call(
        _readout_body,
        grid=(grid,),
        in_specs=[
            pl.BlockSpec((blk, fa), lambda i: (i, 0)),
            pl.BlockSpec((blk, H), lambda i: (i, 0)),
            pl.BlockSpec((1, 1, blk), lambda i: (i, 0, 0)),
            pl.BlockSpec((fa, H), lambda i: (0, 0)),
            pl.BlockSpec((H, H), lambda i: (0, 0)),
            pl.BlockSpec((1, H), lambda i: (0, 0)),
        ],
        out_specs=pl.BlockSpec((MOLP, H), lambda i: (0, 0)),
        out_shape=jax.ShapeDtypeStruct((MOLP, H), jnp.float32),
        scratch_shapes=[pltpu.VMEM((MOLP, H), jnp.float32)],
    )(f_atoms, a_msg, ids3, Wo1, Wo2, b_o)


_MESH = plsc.VectorSubcoreMesh(core_axis_name="c", subcore_axis_name="s")

NAP = 10240
APS = NAP // NC
APT = APS // NS
GROWS = 128
GCHUNKS = APT * 32 // GROWS
GL = 16


def _sum_chunk(rows, k, acc_v):
    for a in range(4):
        def rbody(r, accs):
            return tuple(
                accs[g] + rows[a * 32 + r, pl.ds(g * GL, GL)]
                for g in range(H // GL))
        accs = lax.fori_loop(
            0, 32, rbody,
            tuple(jnp.zeros((GL,), jnp.float32) for _ in range(H // GL)),
            unroll=4)
        for g in range(H // GL):
            acc_v[k * 4 + a, pl.ds(g * GL, GL)] = accs[g]


def _gather_sum_sc(msg_hbm, a2b_hbm, out_hbm,
                   idx_v, rows_a, rows_b, acc_v, sem_a, sem_b):
    c = lax.axis_index("c")
    s = lax.axis_index("s")
    pltpu.sync_copy(
        a2b_hbm.at[pl.ds(c * (APS * 32 // GROWS) + s * GCHUNKS, GCHUNKS)],
        idx_v)
    pltpu.async_copy(msg_hbm.at[idx_v.at[0]], rows_a, sem_a)
    pltpu.async_copy(msg_hbm.at[idx_v.at[1]], rows_b, sem_b)

    def body(j, _):
        k0 = 2 * j
        pltpu.make_async_copy(msg_hbm.at[idx_v.at[k0]], rows_a, sem_a).wait()
        _sum_chunk(rows_a, k0, acc_v)

        @pl.when(k0 + 2 < GCHUNKS)
        def _():
            pltpu.async_copy(msg_hbm.at[idx_v.at[k0 + 2]], rows_a, sem_a)

        pltpu.make_async_copy(msg_hbm.at[idx_v.at[k0 + 1]], rows_b,
                              sem_b).wait()
        _sum_chunk(rows_b, k0 + 1, acc_v)

        @pl.when(k0 + 3 < GCHUNKS)
        def _():
            pltpu.async_copy(msg_hbm.at[idx_v.at[k0 + 3]], rows_b, sem_b)

        return 0

    lax.fori_loop(0, GCHUNKS // 2, body, 0)
    pltpu.sync_copy(acc_v, out_hbm.at[pl.ds(c * APS + s * APT, APT)])


@functools.partial(
    pl.kernel,
    out_type=jax.ShapeDtypeStruct((NAP, H), jnp.float32),
    mesh=_MESH,
    scratch_types=[
        pltpu.VMEM((GCHUNKS, GROWS), jnp.int32),
        pltpu.VMEM((GROWS, H), jnp.float32),
        pltpu.VMEM((GROWS, H), jnp.float32),
        pltpu.VMEM((APT, H), jnp.float32),
        pltpu.SemaphoreType.DMA,
        pltpu.SemaphoreType.DMA,
    ],
)
def _gather_sum(msg_hbm, a2b_hbm, out_hbm,
                idx_v, rows_a, rows_b, acc_v, sem_a, sem_b):
    _gather_sum_sc(msg_hbm, a2b_hbm, out_hbm,
                   idx_v, rows_a, rows_b, acc_v, sem_a, sem_b)


CB = 80


def _diff_chunk(am_v, h_v, out_v):
    def row(r, _):
        for g in range(H // GL):
            sl = pl.ds(g * GL, GL)
            out_v[r, sl] = am_v[r, sl] - h_v[r, sl]
        return 0

    lax.fori_loop(0, CB, row, 0, unroll=2)


def _combine_sc(nb, am_hbm, h_hbm, b2a_hbm, brev_hbm, out_hbm,
                idxa_v, idxr_v, am_a, am_b, h_a, h_b, out_a, out_b,
                sem_list):
    c = lax.axis_index("c")
    s = lax.axis_index("s")
    w = s * NC + c
    bpw = nb // NW
    nchunks = bpw // CB
    sem_aa, sem_ab, sem_ha, sem_hb = sem_list
    pltpu.sync_copy(b2a_hbm.at[w], idxa_v)
    pltpu.sync_copy(brev_hbm.at[w], idxr_v)

    def start_a(k):
        pltpu.async_copy(am_hbm.at[idxa_v.at[k]], am_a, sem_aa)
        pltpu.async_copy(h_hbm.at[idxr_v.at[k]], h_a, sem_ha)

    def start_b(k):
        pltpu.async_copy(am_hbm.at[idxa_v.at[k]], am_b, sem_ab)
        pltpu.async_copy(h_hbm.at[idxr_v.at[k]], h_b, sem_hb)

    start_a(0)
    start_b(1)

    def body(j, _):
        k0 = 2 * j
        base = w * bpw
        pltpu.make_async_copy(am_hbm.at[idxa_v.at[k0]], am_a, sem_aa).wait()
        pltpu.make_async_copy(h_hbm.at[idxr_v.at[k0]], h_a, sem_ha).wait()
        _diff_chunk(am_a, h_a, out_a)
        pltpu.sync_copy(out_a, out_hbm.at[pl.ds(base + k0 * CB, CB)])

        @pl.when(k0 + 2 < nchunks)
        def _():
            start_a(k0 + 2)

        pltpu.make_async_copy(am_hbm.at[idxa_v.at[k0 + 1]], am_b,
                              sem_ab).wait()
        pltpu.make_async_copy(h_hbm.at[idxr_v.at[k0 + 1]], h_b,
                              sem_hb).wait()
        _diff_chunk(am_b, h_b, out_b)
        pltpu.sync_copy(out_b, out_hbm.at[pl.ds(base + (k0 + 1) * CB, CB)])

        @pl.when(k0 + 3 < nchunks)
        def _():
            start_b(k0 + 3)

        return 0

    lax.fori_loop(0, nchunks // 2, body, 0)
    if nchunks % 2 == 1:
        k = nchunks - 1
        pltpu.make_async_copy(am_hbm.at[idxa_v.at[k]], am_a, sem_aa).wait()
        pltpu.make_async_copy(h_hbm.at[idxr_v.at[k]], h_a, sem_ha).wait()
        _diff_chunk(am_a, h_a, out_a)
        pltpu.sync_copy(out_a, out_hbm.at[pl.ds(w * bpw + k * CB, CB)])


def _make_combine(nb):
    nchunks = nb // NW // CB

    @functools.partial(
        pl.kernel,
        out_type=jax.ShapeDtypeStruct((nb, H), jnp.float32),
        mesh=_MESH,
        scratch_types=[
            pltpu.VMEM((nchunks, CB), jnp.int32),
            pltpu.VMEM((nchunks, CB), jnp.int32),
            pltpu.VMEM((CB, H), jnp.float32),
            pltpu.VMEM((CB, H), jnp.float32),
            pltpu.VMEM((CB, H), jnp.float32),
            pltpu.VMEM((CB, H), jnp.float32),
            pltpu.VMEM((CB, H), jnp.float32),
            pltpu.VMEM((CB, H), jnp.float32),
            pltpu.SemaphoreType.DMA,
            pltpu.SemaphoreType.DMA,
            pltpu.SemaphoreType.DMA,
            pltpu.SemaphoreType.DMA,
        ],
    )
    def _combine(am_hbm, h_hbm, b2a_hbm, brev_hbm, out_hbm,
                 idxa_v, idxr_v, am_a, am_b, h_a, h_b, out_a, out_b,
                 sem_aa, sem_ab, sem_ha, sem_hb):
        _combine_sc(nb, am_hbm, h_hbm, b2a_hbm, brev_hbm, out_hbm,
                    idxa_v, idxr_v, am_a, am_b, h_a, h_b, out_a, out_b,
                    (sem_aa, sem_ab, sem_ha, sem_hb))

    return _combine


def kernel(f_atoms, f_bonds, a2b, b2a, b2revb, mol_ids, W_i, W_h, W_o, b_o):
    na, fa_dim = f_atoms.shape
    nb = f_bonds.shape[0]
    maxnb = a2b.shape[1]

    a2b_p = jnp.pad(a2b.astype(jnp.int32), ((0, NAP - na), (0, 0)))
    a2b_rs = a2b_p.reshape(NAP * maxnb // GROWS, GROWS)
    nchunks = nb // NW // CB
    b2a_rs = b2a.astype(jnp.int32).reshape(NW, nchunks, CB)
    brev_rs = b2revb.astype(jnp.int32).reshape(NW, nchunks, CB)
    ids3 = mol_ids.astype(jnp.int32).reshape(5, 1, na // 5)
    Wo1 = W_o[:fa_dim]
    Wo2 = W_o[fa_dim:]
    bo2 = b_o.reshape(1, H)

    combine = _make_combine(nb)

    inp, msg = _input_matmul(f_bonds, W_i, 1600)
    h = _matmul(msg, W_h, 1600)

    for t in range(DEPTH - 1):
        ga = _gather_sum(msg, a2b_rs)
        am_h = _matmul(ga, W_h, 2048)
        pre = combine(am_h, h, b2a_rs, brev_rs)
        if t < DEPTH - 2:
            msg, h = _fuse(inp, pre, W_h, 1600)
        else:
            msg = _fuse_last(inp, pre, 1600)

    ga = _gather_sum(msg, a2b_rs)
    out = _readout(f_atoms, ga[:na], ids3, Wo1, Wo2, bo2, 2000)
    n_mols = 500
    return out[:n_mols]


# speedup vs baseline: 1.0596x; 1.0596x over previous
import functools

import jax
import jax.numpy as jnp
from jax import lax
from jax.experimental import pallas as pl
from jax.experimental.pallas import tpu as pltpu
from jax.experimental.pallas import tpu_sc as plsc

H = 128
DEPTH = 5
NC = 2
NS = 16
NW = NC * NS
F32 = jnp.float32


def _mm_body(x_ref, w_ref, o_ref):
    o_ref[...] = jnp.dot(x_ref[...], w_ref[...], preferred_element_type=F32)


def _matmul(x, w, blk):
    m, k = x.shape
    n = w.shape[1]
    return pl.pallas_call(
        _mm_body,
        grid=(m // blk,),
        in_specs=[
            pl.BlockSpec((blk, k), lambda i: (i, 0)),
            pl.BlockSpec((k, n), lambda i: (0, 0)),
        ],
        out_specs=pl.BlockSpec((blk, n), lambda i: (i, 0)),
        out_shape=jax.ShapeDtypeStruct((m, n), F32),
    )(x, w)


def _mm_relu_body(x_ref, w_ref, inp_ref, msg_ref):
    acc = jnp.dot(x_ref[...], w_ref[...], preferred_element_type=F32)
    inp_ref[...] = acc
    msg_ref[...] = jnp.maximum(acc, 0.0)


def _input_matmul(f_bonds, W_i, blk):
    m, k = f_bonds.shape
    n = W_i.shape[1]
    shp = jax.ShapeDtypeStruct((m, n), F32)
    return pl.pallas_call(
        _mm_relu_body,
        grid=(m // blk,),
        in_specs=[
            pl.BlockSpec((blk, k), lambda i: (i, 0)),
            pl.BlockSpec((k, n), lambda i: (0, 0)),
        ],
        out_specs=[
            pl.BlockSpec((blk, n), lambda i: (i, 0)),
            pl.BlockSpec((blk, n), lambda i: (i, 0)),
        ],
        out_shape=[shp, shp],
    )(f_bonds, W_i)


def _fuse_body(inp_ref, pre_ref, w_ref, msg_ref, h_ref):
    m = jnp.maximum(inp_ref[...] + pre_ref[...], 0.0)
    msg_ref[...] = m
    h_ref[...] = jnp.dot(m, w_ref[...], preferred_element_type=F32)


def _fuse(inp, pre, w, blk):
    m, n = inp.shape
    shp = jax.ShapeDtypeStruct((m, n), F32)
    return pl.pallas_call(
        _fuse_body,
        grid=(m // blk,),
        in_specs=[
            pl.BlockSpec((blk, n), lambda i: (i, 0)),
            pl.BlockSpec((blk, n), lambda i: (i, 0)),
            pl.BlockSpec((n, n), lambda i: (0, 0)),
        ],
        out_specs=[
            pl.BlockSpec((blk, n), lambda i: (i, 0)),
            pl.BlockSpec((blk, n), lambda i: (i, 0)),
        ],
        out_shape=[shp, shp],
    )(inp, pre, w)


def _fuse_last_body(inp_ref, pre_ref, msg_ref):
    msg_ref[...] = jnp.maximum(inp_ref[...] + pre_ref[...], 0.0)


def _fuse_last(inp, pre, blk):
    m, n = inp.shape
    return pl.pallas_call(
        _fuse_last_body,
        grid=(m // blk,),
        in_specs=[
            pl.BlockSpec((blk, n), lambda i: (i, 0)),
            pl.BlockSpec((blk, n), lambda i: (i, 0)),
        ],
        out_specs=pl.BlockSpec((blk, n), lambda i: (i, 0)),
        out_shape=jax.ShapeDtypeStruct((m, n), F32),
    )(inp, pre)


MOLP = 512


def _readout_body(fa_ref, am_ref, ids_ref, wo1_ref, wo2_ref, bo_ref,
                  out_ref, cnt_ref):
    i = pl.program_id(0)
    hidden = jnp.maximum(
        jnp.dot(fa_ref[...], wo1_ref[...], preferred_element_type=F32)
        + jnp.dot(am_ref[...], wo2_ref[...], preferred_element_type=F32)
        + bo_ref[...],
        0.0,
    )
    ids = ids_ref[0, 0, :]
    onehot = (lax.broadcasted_iota(jnp.int32, (MOLP, ids.shape[0]), 0)
              == ids[None, :]).astype(F32)
    part = jnp.dot(onehot, hidden, preferred_element_type=F32)
    cpart = jnp.sum(onehot, axis=1, keepdims=True)

    @pl.when(i == 0)
    def _():
        out_ref[...] = jnp.zeros_like(out_ref)
        cnt_ref[...] = jnp.zeros_like(cnt_ref)

    out_ref[...] += part
    cnt_ref[...] += jnp.broadcast_to(cpart, cnt_ref.shape)

    @pl.when(i == pl.num_programs(0) - 1)
    def _():
        out_ref[...] = out_ref[...] / jnp.maximum(cnt_ref[...], 1.0)


def _readout(f_atoms, a_msg, ids3, Wo1, Wo2, b_o, blk):
    na, fa = f_atoms.shape
    grid = na // blk
    return pl.pallas_call(
        _readout_body,
        grid=(grid,),
        in_specs=[
            pl.BlockSpec((blk, fa), lambda i: (i, 0)),
            pl.BlockSpec((blk, H), lambda i: (i, 0)),
            pl.BlockSpec((1, 1, blk), lambda i: (i, 0, 0)),
            pl.BlockSpec((fa, H), lambda i: (0, 0)),
            pl.BlockSpec((H, H), lambda i: (0, 0)),
            pl.BlockSpec((1, H), lambda i: (0, 0)),
        ],
        out_specs=pl.BlockSpec((MOLP, H), lambda i: (0, 0)),
        out_shape=jax.ShapeDtypeStruct((MOLP, H), F32),
        scratch_shapes=[pltpu.VMEM((MOLP, H), F32)],
    )(f_atoms, a_msg, ids3, Wo1, Wo2, b_o)


_MESH = plsc.VectorSubcoreMesh(core_axis_name="c", subcore_axis_name="s")

NAP = 10240
APS = NAP // NC
APT = APS // NS
GROWS = 128
GCHUNKS = APT * 32 // GROWS
GL = 16


def _sum_chunk(rows, k, acc_v):
    for a in range(4):
        for g in range(H // GL):
            sl = pl.ds(g * GL, GL)
            vals = [rows[a * 32 + r, sl] for r in range(32)]
            while len(vals) > 1:
                vals = [vals[i] + vals[i + 1] for i in range(0, len(vals), 2)]
            acc_v[k * 4 + a, sl] = vals[0]


def _gather_sum_sc(msg_hbm, a2b_hbm, out_hbm,
                   idx_v, rows_a, rows_b, acc_v, sem_a, sem_b):
    c = lax.axis_index("c")
    s = lax.axis_index("s")
    pltpu.sync_copy(
        a2b_hbm.at[pl.ds(c * (APS * 32 // GROWS) + s * GCHUNKS, GCHUNKS)],
        idx_v)
    pltpu.async_copy(msg_hbm.at[idx_v.at[0]], rows_a, sem_a)
    pltpu.async_copy(msg_hbm.at[idx_v.at[1]], rows_b, sem_b)

    def body(j, _):
        k0 = 2 * j
        pltpu.make_async_copy(msg_hbm.at[idx_v.at[k0]], rows_a, sem_a).wait()
        _sum_chunk(rows_a, k0, acc_v)

        @pl.when(k0 + 2 < GCHUNKS)
        def _():
            pltpu.async_copy(msg_hbm.at[idx_v.at[k0 + 2]], rows_a, sem_a)

        pltpu.make_async_copy(msg_hbm.at[idx_v.at[k0 + 1]], rows_b,
                              sem_b).wait()
        _sum_chunk(rows_b, k0 + 1, acc_v)

        @pl.when(k0 + 3 < GCHUNKS)
        def _():
            pltpu.async_copy(msg_hbm.at[idx_v.at[k0 + 3]], rows_b, sem_b)

        return 0

    lax.fori_loop(0, GCHUNKS // 2, body, 0)
    pltpu.sync_copy(acc_v, out_hbm.at[pl.ds(c * APS + s * APT, APT)])


@functools.partial(
    pl.kernel,
    out_type=jax.ShapeDtypeStruct((NAP, H), F32),
    mesh=_MESH,
    scratch_types=[
        pltpu.VMEM((GCHUNKS, GROWS), jnp.int32),
        pltpu.VMEM((GROWS, H), F32),
        pltpu.VMEM((GROWS, H), F32),
        pltpu.VMEM((APT, H), F32),
        pltpu.SemaphoreType.DMA,
        pltpu.SemaphoreType.DMA,
    ],
)
def _gather_sum(msg_hbm, a2b_hbm, out_hbm,
                idx_v, rows_a, rows_b, acc_v, sem_a, sem_b):
    _gather_sum_sc(msg_hbm, a2b_hbm, out_hbm,
                   idx_v, rows_a, rows_b, acc_v, sem_a, sem_b)


CB = 80
NSLOT = 2


def _diff_chunk(am_v, h_v, out_v):
    for r in range(CB):
        for g in range(H // GL):
            sl = pl.ds(g * GL, GL)
            out_v[r, sl] = am_v[r, sl] - h_v[r, sl]


def _combine_sc(nb, am_hbm, h_hbm, b2a_hbm, brev_hbm, out_hbm,
                idxa_v, idxr_v, am_bufs, h_bufs, out_bufs,
                sems_a, sems_h, sems_o):
    c = lax.axis_index("c")
    s = lax.axis_index("s")
    w = s * NC + c
    bpw = nb // NW
    nchunks = bpw // CB
    nbody = nchunks // NSLOT
    pltpu.sync_copy(b2a_hbm.at[w], idxa_v)
    pltpu.sync_copy(brev_hbm.at[w], idxr_v)

    def start(slot, k):
        pltpu.async_copy(am_hbm.at[idxa_v.at[k]], am_bufs[slot],
                         sems_a[slot])
        pltpu.async_copy(h_hbm.at[idxr_v.at[k]], h_bufs[slot],
                         sems_h[slot])

    def wait_gather(slot, k):
        pltpu.make_async_copy(am_hbm.at[idxa_v.at[k]], am_bufs[slot],
                              sems_a[slot]).wait()
        pltpu.make_async_copy(h_hbm.at[idxr_v.at[k]], h_bufs[slot],
                              sems_h[slot]).wait()

    def wait_store(slot):
        pltpu.make_async_copy(out_bufs[slot], out_hbm.at[pl.ds(0, CB)],
                              sems_o[slot]).wait()

    for slot in range(NSLOT):
        start(slot, slot)

    def body(j, _):
        base = w * bpw
        for slot in range(NSLOT):
            k = NSLOT * j + slot
            wait_gather(slot, k)

            @pl.when(j > 0)
            def _():
                wait_store(slot)

            _diff_chunk(am_bufs[slot], h_bufs[slot], out_bufs[slot])
            pltpu.async_copy(out_bufs[slot],
                             out_hbm.at[pl.ds(base + k * CB, CB)],
                             sems_o[slot])

            @pl.when(k + NSLOT < nchunks)
            def _():
                start(slot, k + NSLOT)
        return 0

    lax.fori_loop(0, nbody, body, 0)
    for slot in range(nchunks - NSLOT * nbody):
        k = NSLOT * nbody + slot
        wait_gather(slot, k)
        wait_store(slot)
        _diff_chunk(am_bufs[slot], h_bufs[slot], out_bufs[slot])
        pltpu.async_copy(out_bufs[slot],
                         out_hbm.at[pl.ds(w * bpw + k * CB, CB)],
                         sems_o[slot])
    for slot in range(NSLOT):
        wait_store(slot)


def _make_combine(nb):
    nchunks = nb // NW // CB

    @functools.partial(
        pl.kernel,
        out_type=jax.ShapeDtypeStruct((nb, H), F32),
        mesh=_MESH,
        scratch_types=[
            pltpu.VMEM((nchunks, CB), jnp.int32),
            pltpu.VMEM((nchunks, CB), jnp.int32),
            pltpu.VMEM((CB, H), F32),
            pltpu.VMEM((CB, H), F32),
            pltpu.VMEM((CB, H), F32),
            pltpu.VMEM((CB, H), F32),
            pltpu.VMEM((CB, H), F32),
            pltpu.VMEM((CB, H), F32),
            pltpu.SemaphoreType.DMA,
            pltpu.SemaphoreType.DMA,
            pltpu.SemaphoreType.DMA,
            pltpu.SemaphoreType.DMA,
            pltpu.SemaphoreType.DMA,
            pltpu.SemaphoreType.DMA,
        ],
    )
    def _combine(am_hbm, h_hbm, b2a_hbm, brev_hbm, out_hbm,
                 idxa_v, idxr_v, am0, am1, h0, h1, o0, o1,
                 sa0, sa1, sh0, sh1, so0, so1):
        _combine_sc(nb, am_hbm, h_hbm, b2a_hbm, brev_hbm, out_hbm,
                    idxa_v, idxr_v, (am0, am1), (h0, h1),
                    (o0, o1), (sa0, sa1), (sh0, sh1),
                    (so0, so1))

    return _combine


def kernel(f_atoms, f_bonds, a2b, b2a, b2revb, mol_ids, W_i, W_h, W_o, b_o):
    na, fa_dim = f_atoms.shape
    nb = f_bonds.shape[0]
    maxnb = a2b.shape[1]

    a2b_p = jnp.pad(a2b.astype(jnp.int32), ((0, NAP - na), (0, 0)))
    a2b_rs = a2b_p.reshape(NAP * maxnb // GROWS, GROWS)
    nchunks = nb // NW // CB
    b2a_rs = b2a.astype(jnp.int32).reshape(NW, nchunks, CB)
    brev_rs = b2revb.astype(jnp.int32).reshape(NW, nchunks, CB)
    ids3 = mol_ids.astype(jnp.int32).reshape(5, 1, na // 5)
    Wo1 = W_o[:fa_dim]
    Wo2 = W_o[fa_dim:]
    bo2 = b_o.reshape(1, H)

    combine = _make_combine(nb)

    inp, msg = _input_matmul(f_bonds, W_i, 1600)
    h = _matmul(msg, W_h, 1600)

    for t in range(DEPTH - 1):
        ga = _gather_sum(msg, a2b_rs)
        am_h = _matmul(ga, W_h, 2048)
        pre = combine(am_h, h, b2a_rs, brev_rs)
        if t < DEPTH - 2:
            msg, h = _fuse(inp, pre, W_h, 1600)
        else:
            msg = _fuse_last(inp, pre, 1600)

    ga = _gather_sum(msg, a2b_rs)
    out = _readout(f_atoms, ga[:na], ids3, Wo1, Wo2, bo2, 2000)
    n_mols = 500
    return out[:n_mols]
